# baseline (speedup 1.0000x reference)
import functools

import jax
import jax.numpy as jnp
from jax import lax
from jax.experimental import pallas as pl
from jax.experimental.pallas import tpu as pltpu
from jax.experimental.pallas import tpu_sc as plsc

B, N, K, H, V = 8, 128, 8, 32, 512
HP = H // 2
NW = 32
ROWS_PER_W = (B * N) // NW


def _sc_edge_body(tab_hbm, ed_hbm, out_hbm, tab_v, idx_v, slab_v):
    wid = lax.axis_index("s") * 2 + lax.axis_index("c")
    pltpu.sync_copy(tab_hbm, tab_v)
    lane = lax.iota(jnp.int32, 16)
    eighth = jnp.full((32,), 0.125, jnp.bfloat16)

    def row_body(r, _):
        row = wid * ROWS_PER_W + r
        b = row // N
        i = row % N
        pltpu.sync_copy(ed_hbm.at[row], idx_v)

        def g_body(g):
            d = [idx_v[pl.ds(k * N + g * 16, 16)] for k in range(K)]

            def hp_body(hp):
                a = [
                    plsc.bitcast(
                        plsc.load_gather(tab_v, [d[k] + hp * V]),
                        jnp.bfloat16)
                    for k in range(K)
                ]
                s = ((a[0] + a[1]) + (a[2] + a[3])) + (
                    (a[4] + a[5]) + (a[6] + a[7]))
                plsc.store_scatter(
                    slab_v,
                    [jnp.full((16,), hp, jnp.int32), lane + g * 16],
                    plsc.bitcast(s * eighth, jnp.int32),
                )

            plsc.parallel_loop(0, HP, 1, unroll=4)(hp_body)

        plsc.parallel_loop(0, N // 16, 1)(g_body)
        pltpu.sync_copy(slab_v, out_hbm.at[b, :, i, :])
        return 0

    lax.fori_loop(0, ROWS_PER_W, row_body, 0)


@jax.jit
def _sc_edge(tab_packed, ed_rows):
    mesh = plsc.VectorSubcoreMesh(core_axis_name="c", subcore_axis_name="s")
    return pl.kernel(
        _sc_edge_body,
        out_type=jax.ShapeDtypeStruct((B, HP, N, N), jnp.int32),
        mesh=mesh,
        compiler_params=pltpu.CompilerParams(needs_layout_passes=False),
        scratch_types=[
            pltpu.VMEM((HP * V,), jnp.int32),
            pltpu.VMEM((N * K,), jnp.int32),
            pltpu.VMEM((HP, N), jnp.int32),
        ],
    )(tab_packed, ed_rows)


def _tc_assemble_body(ab_ref, e_ref, o_ref):
    ab = ab_ref[0]
    x = e_ref[0, 0]
    e_even = lax.bitcast_convert_type(x << 16, jnp.float32)
    e_odd = lax.bitcast_convert_type(x & jnp.int32(-65536), jnp.float32)
    base = 2.0 * ab
    row0 = jnp.zeros((1, N), jnp.float32)
    col0 = jnp.zeros((N + 1, 1), jnp.float32)

    def pad(e):
        return jnp.concatenate(
            [col0, jnp.concatenate([row0, e], axis=0)], axis=1)

    o_ref[0, 0] = base + pad(e_even)
    o_ref[0, 1] = base + pad(e_odd)


@jax.jit
def _tc_assemble(attn_bias, edge_packed):
    return pl.pallas_call(
        _tc_assemble_body,
        out_shape=jax.ShapeDtypeStruct((B, H, N + 1, N + 1), jnp.float32),
        grid=(B, HP),
        in_specs=[
            pl.BlockSpec((1, N + 1, N + 1), lambda b, hp: (b, 0, 0)),
            pl.BlockSpec((1, 1, N, N), lambda b, hp: (b, hp, 0, 0)),
        ],
        out_specs=pl.BlockSpec(
            (1, 2, N + 1, N + 1), lambda b, hp: (b, hp, 0, 0)),
    )(attn_bias, edge_packed)


def kernel(attn_bias, edge_data, edge_encoder_weight):
    ed = jnp.transpose(edge_data.astype(jnp.int32), (0, 1, 3, 2)).reshape(
        B * N, K * N)
    w16 = lax.bitcast_convert_type(
        edge_encoder_weight.astype(jnp.bfloat16), jnp.uint16
    ).astype(jnp.uint32)
    packed = (w16[:, 0::2] | (w16[:, 1::2] << 16)).astype(jnp.int32)
    tab = packed.T.reshape(HP * V)
    edge_packed = _sc_edge(tab, ed)
    return _tc_assemble(attn_bias, edge_packed)


# speedup vs baseline: 1.4723x; 1.4723x over previous
import functools

import jax
import jax.numpy as jnp
from jax import lax
from jax.experimental import pallas as pl
from jax.experimental.pallas import tpu as pltpu
from jax.experimental.pallas import tpu_sc as plsc

B, N, K, H, V = 8, 128, 8, 32, 512
HP = H // 2
NW = 32
ROWS_PER_W = (B * N) // NW


def _sc_edge_body(tab_hbm, ed_hbm, out_hbm, tab_v, idx_v, slab_v):
    wid = lax.axis_index("s") * 2 + lax.axis_index("c")
    pltpu.sync_copy(tab_hbm, tab_v)
    lane = lax.iota(jnp.int32, 16)
    eighth = jnp.full((32,), 0.125, jnp.bfloat16)

    def row_body(r, _):
        row = wid * ROWS_PER_W + r
        b = row // N
        i = row % N
        pltpu.sync_copy(ed_hbm.at[row], idx_v)

        def g_body(g):
            d = [idx_v[pl.ds(k * N + g * 16, 16)] for k in range(K)]

            def hp_body(hp):
                a = [
                    plsc.bitcast(
                        plsc.load_gather(tab_v, [d[k] + hp * V]),
                        jnp.bfloat16)
                    for k in range(K)
                ]
                s = ((a[0] + a[1]) + (a[2] + a[3])) + (
                    (a[4] + a[5]) + (a[6] + a[7]))
                plsc.store_scatter(
                    slab_v,
                    [jnp.full((16,), hp, jnp.int32), lane + g * 16],
                    plsc.bitcast(s * eighth, jnp.int32),
                )

            plsc.parallel_loop(0, HP, 1, unroll=4)(hp_body)

        plsc.parallel_loop(0, N // 16, 1)(g_body)
        pltpu.sync_copy(slab_v, out_hbm.at[b, :, i, :])
        return 0

    lax.fori_loop(0, ROWS_PER_W, row_body, 0)


@jax.jit
def _sc_edge(tab_packed, ed_rows):
    mesh = plsc.VectorSubcoreMesh(core_axis_name="c", subcore_axis_name="s")
    return pl.kernel(
        _sc_edge_body,
        out_type=jax.ShapeDtypeStruct((B, HP, N, N), jnp.int32),
        mesh=mesh,
        compiler_params=pltpu.CompilerParams(needs_layout_passes=False),
        scratch_types=[
            pltpu.VMEM((HP * V,), jnp.int32),
            pltpu.VMEM((N * K,), jnp.int32),
            pltpu.VMEM((HP, N), jnp.int32),
        ],
    )(tab_packed, ed_rows)


def _tc_assemble_body(ab_ref, e_ref, o_ref):
    ab = ab_ref[0]
    x = e_ref[0]
    e_even = lax.bitcast_convert_type(x << 16, jnp.float32)
    e_odd = lax.bitcast_convert_type(x & jnp.int32(-65536), jnp.float32)
    row0 = jnp.zeros((HP, 1, N), jnp.float32)
    col0 = jnp.zeros((HP, N + 1, 1), jnp.float32)

    def pad(e):
        return jnp.concatenate(
            [col0, jnp.concatenate([row0, e], axis=1)], axis=2)

    stacked = jnp.concatenate(
        [pad(e_even)[:, None], pad(e_odd)[:, None]], axis=1
    ).reshape(H, N + 1, N + 1)
    o_ref[0] = 2.0 * ab[None] + stacked


@jax.jit
def _tc_assemble(attn_bias, edge_packed):
    return pl.pallas_call(
        _tc_assemble_body,
        out_shape=jax.ShapeDtypeStruct((B, H, N + 1, N + 1), jnp.float32),
        grid=(B,),
        in_specs=[
            pl.BlockSpec((1, N + 1, N + 1), lambda b: (b, 0, 0)),
            pl.BlockSpec((1, HP, N, N), lambda b: (b, 0, 0, 0)),
        ],
        out_specs=pl.BlockSpec(
            (1, H, N + 1, N + 1), lambda b: (b, 0, 0, 0)),
    )(attn_bias, edge_packed)


def kernel(attn_bias, edge_data, edge_encoder_weight):
    ed = jnp.transpose(edge_data.astype(jnp.int32), (0, 1, 3, 2)).reshape(
        B * N, K * N)
    w16 = lax.bitcast_convert_type(
        edge_encoder_weight.astype(jnp.bfloat16), jnp.uint16
    ).astype(jnp.uint32)
    packed = (w16[:, 0::2] | (w16[:, 1::2] << 16)).astype(jnp.int32)
    tab = packed.T.reshape(HP * V)
    edge_packed = _sc_edge(tab, ed)
    return _tc_assemble(attn_bias, edge_packed)
